# speedup vs baseline: 6.7503x; 6.7503x over previous
import functools

import jax
import jax.numpy as jnp
from jax import lax
from jax.experimental import pallas as pl
from jax.experimental.pallas import tpu as pltpu
from jax.experimental.pallas import tpu_sc as plsc

N_NODES = 10000
D = 128
E = 320000
EPS = 1e-5

NC = 2
NS = 16
NW = NC * NS

CH = 128
NCHUNK = E // CH
BASE_CHUNKS = NCHUNK // NW
EXTRA = NCHUNK % NW

ROWS_T = 624
ROWS_LAST = N_NODES - 15 * ROWS_T
ZR = 208

_mesh = plsc.VectorSubcoreMesh(core_axis_name="c", subcore_axis_name="s")


@functools.partial(
    pl.kernel,
    mesh=_mesh,
    out_type=jax.ShapeDtypeStruct((NC, N_NODES, D), jnp.float32),
    scratch_types=[
        pltpu.VMEM((CH,), jnp.int32),
        pltpu.VMEM((CH,), jnp.int32),
        pltpu.VMEM((CH, D), jnp.float32),
        pltpu.VMEM((ZR, D), jnp.float32),
        pltpu.VMEM_SHARED((N_NODES, D), jnp.float32),
        pltpu.SemaphoreType.DMA,
    ],
)
def _sc_agg(x_hbm, col_hbm, row_hbm, part_hbm,
            col_v, row_v, rows_v, zero_v, agg_sh, sem):
    c = lax.axis_index("c")
    s = lax.axis_index("s")
    wid = s * NC + c

    z16 = jnp.zeros((16,), jnp.float32)

    def zbody(i, carry):
        for j in range(D // 16):
            zero_v[i, pl.ds(j * 16, 16)] = z16
        return carry

    lax.fori_loop(0, ZR, zbody, 0)

    row0 = s * ROWS_T
    for k in range(3):
        pltpu.sync_copy(zero_v, agg_sh.at[pl.ds(row0 + k * ZR, ZR)])

    @pl.when(s == NS - 1)
    def _zero_tail():
        pltpu.sync_copy(zero_v.at[pl.ds(0, 16)],
                        agg_sh.at[pl.ds(15 * ROWS_T + 3 * ZR, 16)])

    plsc.subcore_barrier()

    my_chunks = BASE_CHUNKS + jnp.where(wid < EXTRA, 1, 0)
    start_chunk = wid * BASE_CHUNKS + jnp.minimum(wid, EXTRA)

    def ebody(i, carry):
        base = (start_chunk + i) * CH
        pltpu.sync_copy(col_hbm.at[pl.ds(base, CH)], col_v)
        pltpu.sync_copy(row_hbm.at[pl.ds(base, CH)], row_v)
        pltpu.async_copy(x_hbm.at[col_v], rows_v, sem).wait()
        pltpu.sync_copy(rows_v, agg_sh.at[row_v], add=True)
        return carry

    lax.fori_loop(0, my_chunks, ebody, 0)

    plsc.subcore_barrier()

    pltpu.sync_copy(agg_sh.at[pl.ds(row0, ROWS_T)],
                    part_hbm.at[c, pl.ds(row0, ROWS_T)])

    @pl.when(s == NS - 1)
    def _write_tail():
        pltpu.sync_copy(agg_sh.at[pl.ds(15 * ROWS_T + ROWS_T, 16)],
                        part_hbm.at[c, pl.ds(15 * ROWS_T + ROWS_T, 16)])


def _tc_body(part_ref, x_ref, wl_ref, wr_ref, b_ref, gamma_ref, beta_ref,
             out_ref):
    agg = part_ref[0] + part_ref[1]
    h = lax.dot_general(agg, wl_ref[...], (((1,), (1,)), ((), ())),
                        preferred_element_type=jnp.float32)
    h = h + lax.dot_general(x_ref[...], wr_ref[...], (((1,), (1,)), ((), ())),
                            preferred_element_type=jnp.float32)
    h = h + b_ref[...]
    h = jnp.maximum(h, 0.0)
    mean = jnp.mean(h, axis=0, keepdims=True)
    var = jnp.mean((h - mean) ** 2, axis=0, keepdims=True)
    out_ref[...] = (h - mean) * lax.rsqrt(var + EPS) * gamma_ref[...] \
        + beta_ref[...]


def kernel(x, edge_index, W_l, b_l, W_r, b_r, bn_gamma, bn_beta):
    ei = edge_index.astype(jnp.int32)
    row = ei[0]
    col = ei[1]
    part = _sc_agg(x, col, row)
    b = (b_l + b_r).reshape(1, D)
    out = pl.pallas_call(
        _tc_body,
        out_shape=jax.ShapeDtypeStruct((N_NODES, D), jnp.float32),
    )(part, x, W_l, W_r, b, bn_gamma.reshape(1, D), bn_beta.reshape(1, D))
    return out


# speedup vs baseline: 13.1163x; 1.9431x over previous
import functools

import jax
import jax.numpy as jnp
from jax import lax
from jax.experimental import pallas as pl
from jax.experimental.pallas import tpu as pltpu
from jax.experimental.pallas import tpu_sc as plsc

N_NODES = 10000
D = 128
E = 320000
EPS = 1e-5

NC = 2
NS = 16
NW = NC * NS

CH = 128
NCHUNK = E // CH
BASE_CHUNKS = NCHUNK // NW
EXTRA = NCHUNK % NW
MAXC = BASE_CHUNKS + 1

ROWS_T = 624
ROWS_LAST = N_NODES - 15 * ROWS_T

_mesh = plsc.VectorSubcoreMesh(core_axis_name="c", subcore_axis_name="s")


@functools.partial(
    pl.kernel,
    mesh=_mesh,
    out_type=jax.ShapeDtypeStruct((NC, N_NODES, D), jnp.float32),
    scratch_types=[
        pltpu.VMEM((CH,), jnp.int32),
        pltpu.VMEM((CH,), jnp.int32),
        pltpu.VMEM((CH,), jnp.int32),
        pltpu.VMEM((CH,), jnp.int32),
        pltpu.VMEM((CH,), jnp.int32),
        pltpu.VMEM((CH,), jnp.int32),
        pltpu.VMEM((CH,), jnp.int32),
        pltpu.VMEM((CH,), jnp.int32),
        pltpu.VMEM((CH, D), jnp.float32),
        pltpu.VMEM((CH, D), jnp.float32),
        pltpu.VMEM_SHARED((N_NODES, D), jnp.float32),
        pltpu.SemaphoreType.DMA,
        pltpu.SemaphoreType.DMA,
        pltpu.SemaphoreType.DMA,
        pltpu.SemaphoreType.DMA,
        pltpu.SemaphoreType.DMA,
        pltpu.SemaphoreType.DMA,
    ],
)
def _sc_agg(x_hbm, col_hbm, row_hbm, part_hbm,
            cid0, cid1, cid2, cid3, rid0, rid1, rid2, rid3,
            rows0_v, rows1_v, agg_sh,
            gsem0, gsem1, isem0, isem1, isem2, isem3):
    c = lax.axis_index("c")
    s = lax.axis_index("s")
    wid = s * NC + c

    cid = (cid0, cid1, cid2, cid3)
    rid = (rid0, rid1, rid2, rid3)
    isem = (isem0, isem1, isem2, isem3)
    rows = (rows0_v, rows1_v)
    gsem = (gsem0, gsem1)

    z16 = jnp.zeros((16,), jnp.float32)

    def zbody(i, carry):
        for j in range(D // 16):
            rows0_v[i, pl.ds(j * 16, 16)] = z16
        return carry

    lax.fori_loop(0, CH, zbody, 0)

    row0 = s * ROWS_T
    for k in range(4):
        pltpu.sync_copy(rows0_v, agg_sh.at[pl.ds(row0 + k * CH, CH)])
    pltpu.sync_copy(rows0_v.at[pl.ds(0, ROWS_T - 4 * CH)],
                    agg_sh.at[pl.ds(row0 + 4 * CH, ROWS_T - 4 * CH)])

    @pl.when(s == NS - 1)
    def _zero_tail():
        pltpu.sync_copy(rows0_v.at[pl.ds(0, ROWS_LAST - ROWS_T)],
                        agg_sh.at[pl.ds(15 * ROWS_T + ROWS_T,
                                        ROWS_LAST - ROWS_T)])

    plsc.subcore_barrier()

    my_chunks = BASE_CHUNKS + jnp.where(wid < EXTRA, 1, 0)
    start_chunk = wid * BASE_CHUNKS + jnp.minimum(wid, EXTRA)

    def _fire_idx(i, slot):
        base = (start_chunk + i) * CH
        pltpu.async_copy(col_hbm.at[pl.ds(base, CH)], cid[slot], isem[slot])
        pltpu.async_copy(row_hbm.at[pl.ds(base, CH)], rid[slot], isem[slot])

    def _wait_idx(slot):
        pltpu.make_async_copy(col_hbm.at[pl.ds(0, CH)], cid[slot],
                              isem[slot]).wait()
        pltpu.make_async_copy(row_hbm.at[pl.ds(0, CH)], rid[slot],
                              isem[slot]).wait()

    def _fire_gather(slot, rb):
        pltpu.async_copy(x_hbm.at[cid[slot]], rows[rb], gsem[rb])

    def _wait_gather(slot, rb):
        pltpu.make_async_copy(x_hbm.at[cid[slot]], rows[rb],
                              gsem[rb]).wait()

    for j in range(4):
        _fire_idx(j, j)
    for j in range(2):
        _wait_idx(j)
        _fire_gather(j, j)

    def ebody(t, carry):
        for b in range(4):
            i = t * 4 + b
            rb = b % 2

            @pl.when(i < my_chunks)
            def _step():
                _wait_gather(b, rb)
                pltpu.sync_copy(rows[rb], agg_sh.at[rid[b]], add=True)

                @pl.when(i + 2 < my_chunks)
                def _next_gather():
                    _wait_idx((b + 2) % 4)
                    _fire_gather((b + 2) % 4, rb)

                @pl.when(i + 4 < my_chunks)
                def _next_idx():
                    _fire_idx(i + 4, b)
        return carry

    lax.fori_loop(0, (MAXC + 3) // 4, ebody, 0)

    plsc.subcore_barrier()

    pltpu.sync_copy(agg_sh.at[pl.ds(row0, ROWS_T)],
                    part_hbm.at[c, pl.ds(row0, ROWS_T)])

    @pl.when(s == NS - 1)
    def _write_tail():
        pltpu.sync_copy(agg_sh.at[pl.ds(15 * ROWS_T + ROWS_T,
                                        ROWS_LAST - ROWS_T)],
                        part_hbm.at[c, pl.ds(15 * ROWS_T + ROWS_T,
                                             ROWS_LAST - ROWS_T)])


def _tc_body(part_ref, x_ref, wl_ref, wr_ref, b_ref, gamma_ref, beta_ref,
             out_ref):
    agg = part_ref[0] + part_ref[1]
    h = lax.dot_general(agg, wl_ref[...], (((1,), (1,)), ((), ())),
                        preferred_element_type=jnp.float32)
    h = h + lax.dot_general(x_ref[...], wr_ref[...], (((1,), (1,)), ((), ())),
                            preferred_element_type=jnp.float32)
    h = h + b_ref[...]
    h = jnp.maximum(h, 0.0)
    mean = jnp.mean(h, axis=0, keepdims=True)
    var = jnp.mean((h - mean) ** 2, axis=0, keepdims=True)
    out_ref[...] = (h - mean) * lax.rsqrt(var + EPS) * gamma_ref[...] \
        + beta_ref[...]


def kernel(x, edge_index, W_l, b_l, W_r, b_r, bn_gamma, bn_beta):
    ei = edge_index.astype(jnp.int32)
    row = ei[0]
    col = ei[1]
    part = _sc_agg(x, col, row)
    b = (b_l + b_r).reshape(1, D)
    out = pl.pallas_call(
        _tc_body,
        out_shape=jax.ShapeDtypeStruct((N_NODES, D), jnp.float32),
    )(part, x, W_l, W_r, b, bn_gamma.reshape(1, D), bn_beta.reshape(1, D))
    return out
